# baseline (speedup 1.0000x reference)
import functools

import jax
import jax.numpy as jnp
from jax import lax
from jax.experimental import pallas as pl
from jax.experimental.pallas import tpu as pltpu
from jax.experimental.pallas import tpu_sc as plsc

N = 10000
E = 320000
D = 128

_INFO = plsc.get_sparse_core_info()
NC = _INFO.num_cores
NS = _INFO.num_subcores
NW = NC * NS
EPW = E // NW
K = 80
NCHUNK = EPW // K
RPT = N // NS
ZROWS = 125
WPT = 640
WTAIL0 = (NS - 1) * WPT
NUP = NS * WPT

_MESH = plsc.VectorSubcoreMesh(core_axis_name="c", subcore_axis_name="s")


NBUF = 4


def _make_sc_agg(with_deg):
    out_type = [jax.ShapeDtypeStruct((NC, N, D), jnp.float32)]
    scratch = [
        pltpu.VMEM_SHARED((N, D), jnp.float32),
    ]
    per_buf = 7 + (1 if with_deg else 0)
    for _ in range(NBUF):
        scratch += [
            pltpu.VMEM((K,), jnp.int32),
            pltpu.VMEM((K,), jnp.int32),
            pltpu.VMEM((K, D), jnp.float32),
            pltpu.SemaphoreType.DMA,
            pltpu.SemaphoreType.DMA,
            pltpu.SemaphoreType.DMA,
            pltpu.SemaphoreType.DMA,
        ]
        if with_deg:
            scratch.append(pltpu.SemaphoreType.DMA)
    if with_deg:
        out_type.append(jax.ShapeDtypeStruct((NC, NUP), jnp.float32))
        scratch += [
            pltpu.VMEM_SHARED((NUP,), jnp.float32),
            pltpu.VMEM((K,), jnp.float32),
        ]

    @functools.partial(pl.kernel, mesh=_MESH, out_type=out_type,
                       scratch_types=scratch)
    def k(h_hbm, src_hbm, dst_hbm, zf_hbm, zd_hbm, ones_hbm, *rest):
        if with_deg:
            agg_out, deg_out, acc = rest[:3]
            bufs = rest[3:3 + NBUF * per_buf]
            dacc, ones_v = rest[3 + NBUF * per_buf:]
        else:
            agg_out, acc = rest[:2]
            bufs = rest[2:2 + NBUF * per_buf]
        ring = [bufs[i * per_buf:(i + 1) * per_buf] for i in range(NBUF)]
        sidx = [r[0] for r in ring]
        didx = [r[1] for r in ring]
        rows = [r[2] for r in ring]
        ss = [r[3] for r in ring]
        ds = [r[4] for r in ring]
        gs = [r[5] for r in ring]
        scs = [r[6] for r in ring]
        if with_deg:
            dscs = [r[7] for r in ring]

        c = lax.axis_index("c")
        s = lax.axis_index("s")
        wid = s * NC + c
        w0 = pl.multiple_of(s * WPT, 128)

        if with_deg:
            pltpu.sync_copy(ones_hbm, ones_v)
            pltpu.sync_copy(zd_hbm, dacc.at[pl.ds(w0, WPT)])
        for z in range(RPT // ZROWS):
            pltpu.sync_copy(zf_hbm, acc.at[pl.ds(s * RPT + z * ZROWS, ZROWS)])
        plsc.subcore_barrier()

        def start_idx(ci, b):
            base = pl.multiple_of(wid * EPW + ci * K, 8)
            pltpu.make_async_copy(src_hbm.at[pl.ds(base, K)], sidx[b],
                                  ss[b]).start()
            pltpu.make_async_copy(dst_hbm.at[pl.ds(base, K)], didx[b],
                                  ds[b]).start()

        def wait_idx_start_gather(b):
            pltpu.make_async_copy(src_hbm.at[pl.ds(0, K)], sidx[b],
                                  ss[b]).wait()
            pltpu.make_async_copy(h_hbm.at[sidx[b]], rows[b], gs[b]).start()

        def wait_scatter(b):
            pltpu.make_async_copy(rows[b], acc.at[didx[b]], scs[b]).wait()
            if with_deg:
                pltpu.make_async_copy(ones_v, dacc.at[didx[b]],
                                      dscs[b]).wait()

        for b in range(NBUF - 1):
            start_idx(b, b)
        wait_idx_start_gather(0)

        def body(i4, carry):
            for b in range(NBUF):
                ci = NBUF * i4 + b

                @pl.when(ci < NCHUNK)
                def _():
                    pltpu.make_async_copy(h_hbm.at[sidx[b]], rows[b],
                                          gs[b]).wait()

                    @pl.when(ci + 1 < NCHUNK)
                    def _():
                        wait_idx_start_gather((b + 1) % NBUF)

                    pltpu.make_async_copy(dst_hbm.at[pl.ds(0, K)], didx[b],
                                          ds[b]).wait()
                    pltpu.make_async_copy(rows[b], acc.at[didx[b]],
                                          scs[b]).start(add=True)
                    if with_deg:
                        pltpu.make_async_copy(ones_v, dacc.at[didx[b]],
                                              dscs[b]).start(add=True)

                    @pl.when(ci >= 1)
                    def _():
                        wait_scatter((b + NBUF - 1) % NBUF)


                    @pl.when(ci + NBUF - 1 < NCHUNK)
                    def _():
                        start_idx(ci + NBUF - 1, (b + NBUF - 1) % NBUF)

            return carry

        lax.fori_loop(0, (NCHUNK + NBUF - 1) // NBUF, body, 0)
        wait_scatter((NCHUNK - 1) % NBUF)
        plsc.subcore_barrier()

        if with_deg:
            pltpu.sync_copy(dacc.at[pl.ds(w0, WPT)],
                            deg_out.at[c, pl.ds(w0, WPT)])

        @pl.when(s < NS - 1)
        def _write_full():
            pltpu.sync_copy(acc.at[pl.ds(w0, WPT)],
                            agg_out.at[c, pl.ds(w0, WPT)])

        @pl.when(s == NS - 1)
        def _write_tail():
            pltpu.sync_copy(acc.at[pl.ds(WTAIL0, N - WTAIL0)],
                            agg_out.at[c, pl.ds(WTAIL0, N - WTAIL0)])

    return k


_sc_agg_deg = _make_sc_agg(True)
_sc_agg_only = _make_sc_agg(False)


def _tc_layer_body(relu, p_ref, d_ref, h_ref, wl_ref, wr_ref, b_ref, o_ref):
    agg = p_ref[0] + p_ref[1]
    deg = d_ref[...]
    mean = agg / jnp.maximum(deg, 1.0)
    out = (jnp.dot(mean, wl_ref[...], preferred_element_type=jnp.float32)
           + jnp.dot(h_ref[...], wr_ref[...], preferred_element_type=jnp.float32)
           + b_ref[...])
    if relu:
        out = jnp.maximum(out, 0.0)
    o_ref[...] = out


def _tc_layer(aggp, deg_col, h, Wl, Wr, b, relu):
    BN = 1000
    grid = (N // BN,)
    return pl.pallas_call(
        functools.partial(_tc_layer_body, relu),
        grid=grid,
        in_specs=[
            pl.BlockSpec((NC, BN, D), lambda i: (0, i, 0)),
            pl.BlockSpec((BN, 1), lambda i: (i, 0)),
            pl.BlockSpec((BN, D), lambda i: (i, 0)),
            pl.BlockSpec((D, D), lambda i: (0, 0)),
            pl.BlockSpec((D, D), lambda i: (0, 0)),
            pl.BlockSpec((1, D), lambda i: (0, 0)),
        ],
        out_specs=pl.BlockSpec((BN, D), lambda i: (i, 0)),
        out_shape=jax.ShapeDtypeStruct((N, D), jnp.float32),
    )(aggp, deg_col, h, Wl, Wr, b.reshape(1, D))


def kernel(x, edge_index, Wl0, Wr0, b0, Wl1, Wr1, b1):
    src = edge_index[0]
    dst = edge_index[1]
    zf = jnp.zeros((ZROWS, D), jnp.float32)
    zd = jnp.zeros((WPT,), jnp.float32)
    ones = jnp.ones((K,), jnp.float32)

    aggp0, degp = _sc_agg_deg(x, src, dst, zf, zd, ones)
    deg_col = (degp[0, :N] + degp[1, :N]).reshape(N, 1)
    h1 = _tc_layer(aggp0, deg_col, x, Wl0, Wr0, b0, relu=True)
    (aggp1,) = _sc_agg_only(h1, src, dst, zf, zd, ones)
    out = _tc_layer(aggp1, deg_col, h1, Wl1, Wr1, b1, relu=False)
    return out


# speedup vs baseline: 1.1034x; 1.1034x over previous
import functools

import jax
import jax.numpy as jnp
from jax import lax
from jax.experimental import pallas as pl
from jax.experimental.pallas import tpu as pltpu
from jax.experimental.pallas import tpu_sc as plsc

N = 10000
E = 320000
D = 128

_INFO = plsc.get_sparse_core_info()
NC = _INFO.num_cores
NS = _INFO.num_subcores
NW = NC * NS
EPW = E // NW
K = 40
NCHUNK = EPW // K
RPT = N // NS
ZROWS = 125
WPT = 640
WTAIL0 = (NS - 1) * WPT
NUP = NS * WPT

_MESH = plsc.VectorSubcoreMesh(core_axis_name="c", subcore_axis_name="s")


NBUF = 8
GL = NBUF - 2


def _make_sc_agg(with_deg):
    out_type = [jax.ShapeDtypeStruct((NC, N, D), jnp.float32)]
    scratch = [
        pltpu.VMEM_SHARED((N, D), jnp.float32),
    ]
    per_buf = 6
    for _ in range(NBUF):
        scratch += [
            pltpu.VMEM((K,), jnp.int32),
            pltpu.VMEM((K,), jnp.int32),
            pltpu.VMEM((K, D), jnp.float32),
            pltpu.SemaphoreType.DMA,
            pltpu.SemaphoreType.DMA,
            pltpu.SemaphoreType.DMA,
        ]
    if with_deg:
        out_type.append(jax.ShapeDtypeStruct((NC, NUP), jnp.float32))
        scratch += [
            pltpu.VMEM_SHARED((NUP,), jnp.float32),
            pltpu.VMEM((K,), jnp.float32),
        ]

    @functools.partial(pl.kernel, mesh=_MESH, out_type=out_type,
                       scratch_types=scratch)
    def k(h_hbm, src_hbm, dst_hbm, zf_hbm, zd_hbm, ones_hbm, *rest):
        if with_deg:
            agg_out, deg_out, acc = rest[:3]
            bufs = rest[3:3 + NBUF * per_buf]
            dacc, ones_v = rest[3 + NBUF * per_buf:]
        else:
            agg_out, acc = rest[:2]
            bufs = rest[2:2 + NBUF * per_buf]
        ring = [bufs[i * per_buf:(i + 1) * per_buf] for i in range(NBUF)]
        sidx = [r[0] for r in ring]
        didx = [r[1] for r in ring]
        rows = [r[2] for r in ring]
        isem = [r[3] for r in ring]
        gs = [r[4] for r in ring]
        scs = [r[5] for r in ring]

        c = lax.axis_index("c")
        s = lax.axis_index("s")
        wid = s * NC + c
        w0 = pl.multiple_of(s * WPT, 128)

        if with_deg:
            pltpu.sync_copy(ones_hbm, ones_v)
            pltpu.sync_copy(zd_hbm, dacc.at[pl.ds(w0, WPT)])
        for z in range(RPT // ZROWS):
            pltpu.sync_copy(zf_hbm, acc.at[pl.ds(s * RPT + z * ZROWS, ZROWS)])
        plsc.subcore_barrier()

        def start_idx(ci, b):
            base = pl.multiple_of(wid * EPW + ci * K, 8)
            pltpu.make_async_copy(src_hbm.at[pl.ds(base, K)], sidx[b],
                                  isem[b]).start()
            pltpu.make_async_copy(dst_hbm.at[pl.ds(base, K)], didx[b],
                                  isem[b]).start()

        def wait_idx_start_gather(b):
            pltpu.make_async_copy(src_hbm.at[pl.ds(0, K)], sidx[b],
                                  isem[b]).wait()
            pltpu.make_async_copy(dst_hbm.at[pl.ds(0, K)], didx[b],
                                  isem[b]).wait()
            pltpu.make_async_copy(h_hbm.at[sidx[b]], rows[b], gs[b]).start()

        def start_scatter(b):
            pltpu.make_async_copy(rows[b], acc.at[didx[b]],
                                  scs[b]).start(add=True)
            if with_deg:
                pltpu.make_async_copy(ones_v, dacc.at[didx[b]],
                                      scs[b]).start(add=True)

        def wait_scatter(b):
            pltpu.make_async_copy(rows[b], acc.at[didx[b]], scs[b]).wait()
            if with_deg:
                pltpu.make_async_copy(ones_v, dacc.at[didx[b]],
                                      scs[b]).wait()

        for b in range(NBUF - 1):
            start_idx(b, b)
        for b in range(GL):
            wait_idx_start_gather(b)

        def body(i8, carry):
            for b in range(NBUF):
                ci = NBUF * i8 + b

                @pl.when(ci < NCHUNK)
                def _():
                    @pl.when(ci + GL < NCHUNK)
                    def _():
                        wait_idx_start_gather((b + GL) % NBUF)

                    pltpu.make_async_copy(h_hbm.at[sidx[b]], rows[b],
                                          gs[b]).wait()
                    start_scatter(b)

                    @pl.when(ci >= 1)
                    def _():
                        wait_scatter((b + NBUF - 1) % NBUF)

                    @pl.when(ci + NBUF - 1 < NCHUNK)
                    def _():
                        start_idx(ci + NBUF - 1, (b + NBUF - 1) % NBUF)

            return carry

        lax.fori_loop(0, (NCHUNK + NBUF - 1) // NBUF, body, 0)
        wait_scatter((NCHUNK - 1) % NBUF)
        plsc.subcore_barrier()

        if with_deg:
            pltpu.sync_copy(dacc.at[pl.ds(w0, WPT)],
                            deg_out.at[c, pl.ds(w0, WPT)])

        @pl.when(s < NS - 1)
        def _write_full():
            pltpu.sync_copy(acc.at[pl.ds(w0, WPT)],
                            agg_out.at[c, pl.ds(w0, WPT)])

        @pl.when(s == NS - 1)
        def _write_tail():
            pltpu.sync_copy(acc.at[pl.ds(WTAIL0, N - WTAIL0)],
                            agg_out.at[c, pl.ds(WTAIL0, N - WTAIL0)])

    return k


_sc_agg_deg = _make_sc_agg(True)
_sc_agg_only = _make_sc_agg(False)


def _tc_layer_body(relu, p_ref, d_ref, h_ref, wl_ref, wr_ref, b_ref, o_ref):
    agg = p_ref[0] + p_ref[1]
    deg = d_ref[...]
    mean = agg / jnp.maximum(deg, 1.0)
    out = (jnp.dot(mean, wl_ref[...], preferred_element_type=jnp.float32)
           + jnp.dot(h_ref[...], wr_ref[...], preferred_element_type=jnp.float32)
           + b_ref[...])
    if relu:
        out = jnp.maximum(out, 0.0)
    o_ref[...] = out


def _tc_layer(aggp, deg_col, h, Wl, Wr, b, relu):
    BN = 1000
    grid = (N // BN,)
    return pl.pallas_call(
        functools.partial(_tc_layer_body, relu),
        grid=grid,
        in_specs=[
            pl.BlockSpec((NC, BN, D), lambda i: (0, i, 0)),
            pl.BlockSpec((BN, 1), lambda i: (i, 0)),
            pl.BlockSpec((BN, D), lambda i: (i, 0)),
            pl.BlockSpec((D, D), lambda i: (0, 0)),
            pl.BlockSpec((D, D), lambda i: (0, 0)),
            pl.BlockSpec((1, D), lambda i: (0, 0)),
        ],
        out_specs=pl.BlockSpec((BN, D), lambda i: (i, 0)),
        out_shape=jax.ShapeDtypeStruct((N, D), jnp.float32),
    )(aggp, deg_col, h, Wl, Wr, b.reshape(1, D))


def kernel(x, edge_index, Wl0, Wr0, b0, Wl1, Wr1, b1):
    src = edge_index[0]
    dst = edge_index[1]
    zf = jnp.zeros((ZROWS, D), jnp.float32)
    zd = jnp.zeros((WPT,), jnp.float32)
    ones = jnp.ones((K,), jnp.float32)

    aggp0, degp = _sc_agg_deg(x, src, dst, zf, zd, ones)
    deg_col = (degp[0, :N] + degp[1, :N]).reshape(N, 1)
    h1 = _tc_layer(aggp0, deg_col, x, Wl0, Wr0, b0, relu=True)
    (aggp1,) = _sc_agg_only(h1, src, dst, zf, zd, ones)
    out = _tc_layer(aggp1, deg_col, h1, Wl1, Wr1, b1, relu=False)
    return out


# speedup vs baseline: 1.1129x; 1.0087x over previous
import functools

import jax
import jax.numpy as jnp
from jax import lax
from jax.experimental import pallas as pl
from jax.experimental.pallas import tpu as pltpu
from jax.experimental.pallas import tpu_sc as plsc

N = 10000
E = 320000
D = 128

_INFO = plsc.get_sparse_core_info()
NC = _INFO.num_cores
NS = _INFO.num_subcores
NW = NC * NS
EPW = E // NW
K = 40
NCHUNK = EPW // K
RPT = N // NS
ZROWS = 125
WPT = 640
WTAIL0 = (NS - 1) * WPT
NUP = NS * WPT

_MESH = plsc.VectorSubcoreMesh(core_axis_name="c", subcore_axis_name="s")


NBUF = 8
GL = NBUF - 2


def _make_sc_agg(with_deg):
    out_type = [jax.ShapeDtypeStruct((NC, N, D), jnp.float32)]
    scratch = [
        pltpu.VMEM_SHARED((N, D), jnp.float32),
    ]
    per_buf = 6
    for _ in range(NBUF):
        scratch += [
            pltpu.VMEM((K,), jnp.int32),
            pltpu.VMEM((K,), jnp.int32),
            pltpu.VMEM((K, D), jnp.float32),
            pltpu.SemaphoreType.DMA,
            pltpu.SemaphoreType.DMA,
            pltpu.SemaphoreType.DMA,
        ]
    if with_deg:
        out_type.append(jax.ShapeDtypeStruct((NC, NUP), jnp.float32))
        scratch += [
            pltpu.VMEM_SHARED((NUP,), jnp.float32),
            pltpu.VMEM((K,), jnp.float32),
        ]

    @functools.partial(pl.kernel, mesh=_MESH, out_type=out_type,
                       scratch_types=scratch)
    def k(h_hbm, src_hbm, dst_hbm, zf_hbm, zd_hbm, ones_hbm, *rest):
        if with_deg:
            agg_out, deg_out, acc = rest[:3]
            bufs = rest[3:3 + NBUF * per_buf]
            dacc, ones_v = rest[3 + NBUF * per_buf:]
        else:
            agg_out, acc = rest[:2]
            bufs = rest[2:2 + NBUF * per_buf]
        ring = [bufs[i * per_buf:(i + 1) * per_buf] for i in range(NBUF)]
        sidx = [r[0] for r in ring]
        didx = [r[1] for r in ring]
        rows = [r[2] for r in ring]
        isem = [r[3] for r in ring]
        gs = [r[4] for r in ring]
        scs = [r[5] for r in ring]

        c = lax.axis_index("c")
        s = lax.axis_index("s")
        wid = s * NC + c
        w0 = pl.multiple_of(s * WPT, 128)

        zsem = scs[0]
        if with_deg:
            pltpu.make_async_copy(ones_hbm, ones_v, zsem).start()
            pltpu.make_async_copy(zd_hbm, dacc.at[pl.ds(w0, WPT)],
                                  zsem).start()
        for z in range(RPT // ZROWS):
            pltpu.make_async_copy(
                zf_hbm, acc.at[pl.ds(s * RPT + z * ZROWS, ZROWS)],
                zsem).start()
        def drain_zero():
            if with_deg:
                pltpu.make_async_copy(ones_hbm, ones_v, zsem).wait()
                pltpu.make_async_copy(zd_hbm, dacc.at[pl.ds(w0, WPT)],
                                      zsem).wait()
            for z in range(RPT // ZROWS):
                pltpu.make_async_copy(
                    zf_hbm, acc.at[pl.ds(s * RPT + z * ZROWS, ZROWS)],
                    zsem).wait()

        def start_idx(ci, b):
            base = pl.multiple_of(wid * EPW + ci * K, 8)
            pltpu.make_async_copy(src_hbm.at[pl.ds(base, K)], sidx[b],
                                  isem[b]).start()
            pltpu.make_async_copy(dst_hbm.at[pl.ds(base, K)], didx[b],
                                  isem[b]).start()

        def wait_idx_start_gather(b):
            pltpu.make_async_copy(src_hbm.at[pl.ds(0, K)], sidx[b],
                                  isem[b]).wait()
            pltpu.make_async_copy(dst_hbm.at[pl.ds(0, K)], didx[b],
                                  isem[b]).wait()
            pltpu.make_async_copy(h_hbm.at[sidx[b]], rows[b], gs[b]).start()

        def start_scatter(b):
            pltpu.make_async_copy(rows[b], acc.at[didx[b]],
                                  scs[b]).start(add=True)
            if with_deg:
                pltpu.make_async_copy(ones_v, dacc.at[didx[b]],
                                      scs[b]).start(add=True)

        def wait_scatter(b):
            pltpu.make_async_copy(rows[b], acc.at[didx[b]], scs[b]).wait()
            if with_deg:
                pltpu.make_async_copy(ones_v, dacc.at[didx[b]],
                                      scs[b]).wait()

        for b in range(NBUF - 1):
            start_idx(b, b)
        for b in range(GL):
            wait_idx_start_gather(b)
        drain_zero()
        plsc.subcore_barrier()

        def body(i8, carry):
            for b in range(NBUF):
                ci = NBUF * i8 + b

                @pl.when(ci < NCHUNK)
                def _():
                    @pl.when(ci + GL < NCHUNK)
                    def _():
                        wait_idx_start_gather((b + GL) % NBUF)

                    pltpu.make_async_copy(h_hbm.at[sidx[b]], rows[b],
                                          gs[b]).wait()
                    start_scatter(b)

                    @pl.when(ci >= 1)
                    def _():
                        wait_scatter((b + NBUF - 1) % NBUF)

                    @pl.when(ci + NBUF - 1 < NCHUNK)
                    def _():
                        start_idx(ci + NBUF - 1, (b + NBUF - 1) % NBUF)

            return carry

        lax.fori_loop(0, (NCHUNK + NBUF - 1) // NBUF, body, 0)
        wait_scatter((NCHUNK - 1) % NBUF)
        plsc.subcore_barrier()

        if with_deg:
            pltpu.sync_copy(dacc.at[pl.ds(w0, WPT)],
                            deg_out.at[c, pl.ds(w0, WPT)])

        @pl.when(s < NS - 1)
        def _write_full():
            pltpu.sync_copy(acc.at[pl.ds(w0, WPT)],
                            agg_out.at[c, pl.ds(w0, WPT)])

        @pl.when(s == NS - 1)
        def _write_tail():
            pltpu.sync_copy(acc.at[pl.ds(WTAIL0, N - WTAIL0)],
                            agg_out.at[c, pl.ds(WTAIL0, N - WTAIL0)])

    return k


_sc_agg_deg = _make_sc_agg(True)
_sc_agg_only = _make_sc_agg(False)


def _tc_layer_body(relu, p_ref, d_ref, h_ref, wl_ref, wr_ref, b_ref, o_ref):
    agg = p_ref[0] + p_ref[1]
    deg = d_ref[...]
    mean = agg / jnp.maximum(deg, 1.0)
    out = (jnp.dot(mean, wl_ref[...], preferred_element_type=jnp.float32)
           + jnp.dot(h_ref[...], wr_ref[...], preferred_element_type=jnp.float32)
           + b_ref[...])
    if relu:
        out = jnp.maximum(out, 0.0)
    o_ref[...] = out


def _tc_layer(aggp, deg_col, h, Wl, Wr, b, relu):
    BN = 1000
    grid = (N // BN,)
    return pl.pallas_call(
        functools.partial(_tc_layer_body, relu),
        grid=grid,
        in_specs=[
            pl.BlockSpec((NC, BN, D), lambda i: (0, i, 0)),
            pl.BlockSpec((BN, 1), lambda i: (i, 0)),
            pl.BlockSpec((BN, D), lambda i: (i, 0)),
            pl.BlockSpec((D, D), lambda i: (0, 0)),
            pl.BlockSpec((D, D), lambda i: (0, 0)),
            pl.BlockSpec((1, D), lambda i: (0, 0)),
        ],
        out_specs=pl.BlockSpec((BN, D), lambda i: (i, 0)),
        out_shape=jax.ShapeDtypeStruct((N, D), jnp.float32),
    )(aggp, deg_col, h, Wl, Wr, b.reshape(1, D))


def kernel(x, edge_index, Wl0, Wr0, b0, Wl1, Wr1, b1):
    src = edge_index[0]
    dst = edge_index[1]
    zf = jnp.zeros((ZROWS, D), jnp.float32)
    zd = jnp.zeros((WPT,), jnp.float32)
    ones = jnp.ones((K,), jnp.float32)

    aggp0, degp = _sc_agg_deg(x, src, dst, zf, zd, ones)
    deg_col = (degp[0, :N] + degp[1, :N]).reshape(N, 1)
    h1 = _tc_layer(aggp0, deg_col, x, Wl0, Wr0, b0, relu=True)
    (aggp1,) = _sc_agg_only(h1, src, dst, zf, zd, ones)
    out = _tc_layer(aggp1, deg_col, h1, Wl1, Wr1, b1, relu=False)
    return out


# speedup vs baseline: 1.1323x; 1.0174x over previous
import functools

import jax
import jax.numpy as jnp
from jax import lax
from jax.experimental import pallas as pl
from jax.experimental.pallas import tpu as pltpu
from jax.experimental.pallas import tpu_sc as plsc

N = 10000
E = 320000
D = 128

_INFO = plsc.get_sparse_core_info()
NC = _INFO.num_cores
NS = _INFO.num_subcores
NW = NC * NS
EPW = E // NW
K = 40
NCHUNK = EPW // K
RPT = N // NS
ZROWS = 125
WPT = 640
WTAIL0 = (NS - 1) * WPT
NUP = NS * WPT

_MESH = plsc.VectorSubcoreMesh(core_axis_name="c", subcore_axis_name="s")


NBUF = 8
GL = NBUF - 2


def _make_sc_agg(with_deg):
    out_type = [jax.ShapeDtypeStruct((NC, N, D), jnp.float32)]
    scratch = [
        pltpu.VMEM_SHARED((N, D), jnp.float32),
    ]
    per_buf = 6
    for _ in range(NBUF):
        scratch += [
            pltpu.VMEM((K,), jnp.int32),
            pltpu.VMEM((K,), jnp.int32),
            pltpu.VMEM((K, D), jnp.float32),
            pltpu.SemaphoreType.DMA,
            pltpu.SemaphoreType.DMA,
            pltpu.SemaphoreType.DMA,
        ]
    if with_deg:
        out_type.append(jax.ShapeDtypeStruct((NC, NUP), jnp.float32))
        scratch += [
            pltpu.VMEM_SHARED((NUP,), jnp.float32),
            pltpu.VMEM((K,), jnp.float32),
        ]

    @functools.partial(pl.kernel, mesh=_MESH, out_type=out_type,
                       scratch_types=scratch)
    def k(h_hbm, src_hbm, dst_hbm, zf_hbm, zd_hbm, ones_hbm, *rest):
        if with_deg:
            agg_out, deg_out, acc = rest[:3]
            bufs = rest[3:3 + NBUF * per_buf]
            dacc, ones_v = rest[3 + NBUF * per_buf:]
        else:
            agg_out, acc = rest[:2]
            bufs = rest[2:2 + NBUF * per_buf]
        ring = [bufs[i * per_buf:(i + 1) * per_buf] for i in range(NBUF)]
        sidx = [r[0] for r in ring]
        didx = [r[1] for r in ring]
        rows = [r[2] for r in ring]
        isem = [r[3] for r in ring]
        gs = [r[4] for r in ring]
        scs = [r[5] for r in ring]

        c = lax.axis_index("c")
        s = lax.axis_index("s")
        wid = s * NC + c
        w0 = pl.multiple_of(s * WPT, 128)

        zsem = scs[0]
        if with_deg:
            pltpu.make_async_copy(ones_hbm, ones_v, zsem).start()
            pltpu.make_async_copy(zd_hbm, dacc.at[pl.ds(w0, WPT)],
                                  zsem).start()
        for z in range(RPT // ZROWS):
            pltpu.make_async_copy(
                zf_hbm, acc.at[pl.ds(s * RPT + z * ZROWS, ZROWS)],
                zsem).start()
        def drain_zero():
            if with_deg:
                pltpu.make_async_copy(ones_hbm, ones_v, zsem).wait()
                pltpu.make_async_copy(zd_hbm, dacc.at[pl.ds(w0, WPT)],
                                      zsem).wait()
            for z in range(RPT // ZROWS):
                pltpu.make_async_copy(
                    zf_hbm, acc.at[pl.ds(s * RPT + z * ZROWS, ZROWS)],
                    zsem).wait()

        def start_idx(ci, b):
            base = pl.multiple_of(wid * EPW + ci * K, 8)
            pltpu.make_async_copy(src_hbm.at[pl.ds(base, K)], sidx[b],
                                  isem[b]).start()
            pltpu.make_async_copy(dst_hbm.at[pl.ds(base, K)], didx[b],
                                  isem[b]).start()

        def wait_idx_start_gather(b):
            pltpu.make_async_copy(src_hbm.at[pl.ds(0, K)], sidx[b],
                                  isem[b]).wait()
            pltpu.make_async_copy(dst_hbm.at[pl.ds(0, K)], didx[b],
                                  isem[b]).wait()
            pltpu.make_async_copy(h_hbm.at[sidx[b]], rows[b], gs[b]).start()

        def start_scatter(b):
            pltpu.make_async_copy(rows[b], acc.at[didx[b]],
                                  scs[b]).start(add=True)
            if with_deg:
                pltpu.make_async_copy(ones_v, dacc.at[didx[b]],
                                      scs[b]).start(add=True)

        def wait_scatter(b):
            pltpu.make_async_copy(rows[b], acc.at[didx[b]], scs[b]).wait()
            if with_deg:
                pltpu.make_async_copy(ones_v, dacc.at[didx[b]],
                                      scs[b]).wait()

        for b in range(NBUF - 1):
            start_idx(b, b)
        for b in range(GL):
            wait_idx_start_gather(b)
        drain_zero()
        plsc.subcore_barrier()

        def body(i8, carry):
            for b in range(NBUF):
                ci = NBUF * i8 + b

                @pl.when(ci < NCHUNK)
                def _():
                    @pl.when(ci + GL < NCHUNK)
                    def _():
                        wait_idx_start_gather((b + GL) % NBUF)

                    pltpu.make_async_copy(h_hbm.at[sidx[b]], rows[b],
                                          gs[b]).wait()
                    start_scatter(b)

                    @pl.when(ci >= 1)
                    def _():
                        wait_scatter((b + NBUF - 1) % NBUF)

                    @pl.when(ci + NBUF - 1 < NCHUNK)
                    def _():
                        start_idx(ci + NBUF - 1, (b + NBUF - 1) % NBUF)

            return carry

        lax.fori_loop(0, (NCHUNK + NBUF - 1) // NBUF, body, 0)
        wait_scatter((NCHUNK - 1) % NBUF)
        plsc.subcore_barrier()

        if with_deg:
            pltpu.sync_copy(dacc.at[pl.ds(w0, WPT)],
                            deg_out.at[c, pl.ds(w0, WPT)])

        @pl.when(s < NS - 1)
        def _write_full():
            pltpu.sync_copy(acc.at[pl.ds(w0, WPT)],
                            agg_out.at[c, pl.ds(w0, WPT)])

        @pl.when(s == NS - 1)
        def _write_tail():
            pltpu.sync_copy(acc.at[pl.ds(WTAIL0, N - WTAIL0)],
                            agg_out.at[c, pl.ds(WTAIL0, N - WTAIL0)])

    return k


_sc_agg_deg = _make_sc_agg(True)
_sc_agg_only = _make_sc_agg(False)


_BN = 1000


def _tc_self_body(h_ref, w_ref, b_ref, o_ref):
    o_ref[...] = (jnp.dot(h_ref[...], w_ref[...],
                          preferred_element_type=jnp.float32) + b_ref[...])


def _tc_self(h, W, b):
    return pl.pallas_call(
        _tc_self_body,
        grid=(N // _BN,),
        in_specs=[
            pl.BlockSpec((_BN, D), lambda i: (i, 0)),
            pl.BlockSpec((D, D), lambda i: (0, 0)),
            pl.BlockSpec((1, D), lambda i: (0, 0)),
        ],
        out_specs=pl.BlockSpec((_BN, D), lambda i: (i, 0)),
        out_shape=jax.ShapeDtypeStruct((N, D), jnp.float32),
    )(h, W, b.reshape(1, D))


def _tc_combine_body(relu, p_ref, d_ref, s_ref, wl_ref, o_ref):
    agg = p_ref[0] + p_ref[1]
    mean = agg / jnp.maximum(d_ref[...], 1.0)
    out = jnp.dot(mean, wl_ref[...],
                  preferred_element_type=jnp.float32) + s_ref[...]
    if relu:
        out = jnp.maximum(out, 0.0)
    o_ref[...] = out


def _tc_combine(aggp, deg_col, selfp, Wl, relu):
    return pl.pallas_call(
        functools.partial(_tc_combine_body, relu),
        grid=(N // _BN,),
        in_specs=[
            pl.BlockSpec((NC, _BN, D), lambda i: (0, i, 0)),
            pl.BlockSpec((_BN, 1), lambda i: (i, 0)),
            pl.BlockSpec((_BN, D), lambda i: (i, 0)),
            pl.BlockSpec((D, D), lambda i: (0, 0)),
        ],
        out_specs=pl.BlockSpec((_BN, D), lambda i: (i, 0)),
        out_shape=jax.ShapeDtypeStruct((N, D), jnp.float32),
    )(aggp, deg_col, selfp, Wl)


def kernel(x, edge_index, Wl0, Wr0, b0, Wl1, Wr1, b1):
    src = edge_index[0]
    dst = edge_index[1]
    zf = jnp.zeros((ZROWS, D), jnp.float32)
    zd = jnp.zeros((WPT,), jnp.float32)
    ones = jnp.ones((K,), jnp.float32)

    aggp0, degp = _sc_agg_deg(x, src, dst, zf, zd, ones)
    self0 = _tc_self(x, Wr0, b0)
    deg_col = (degp[0, :N] + degp[1, :N]).reshape(N, 1)
    h1 = _tc_combine(aggp0, deg_col, self0, Wl0, relu=True)
    (aggp1,) = _sc_agg_only(h1, src, dst, zf, zd, ones)
    self1 = _tc_self(h1, Wr1, b1)
    out = _tc_combine(aggp1, deg_col, self1, Wl1, relu=False)
    return out


# speedup vs baseline: 1.4371x; 1.2692x over previous
import functools

import jax
import jax.numpy as jnp
from jax import lax
from jax.experimental import pallas as pl
from jax.experimental.pallas import tpu as pltpu
from jax.experimental.pallas import tpu_sc as plsc

N = 10000
E = 320000
D = 128

_INFO = plsc.get_sparse_core_info()
NC = _INFO.num_cores
NS = _INFO.num_subcores
NW = NC * NS
EPW = E // NW
K = 80
NCHUNK = EPW // K
RPT = N // NS
ZROWS = 125
WPT = 640
WTAIL0 = (NS - 1) * WPT
NUP = NS * WPT

_MESH = plsc.VectorSubcoreMesh(core_axis_name="c", subcore_axis_name="s")


NBUF = 4
GL = NBUF - 2


def _make_sc_agg(with_deg):
    out_type = [jax.ShapeDtypeStruct((NC, N, D), jnp.float32)]
    scratch = [
        pltpu.VMEM_SHARED((N, D), jnp.float32),
    ]
    per_buf = 6
    for _ in range(NBUF):
        scratch += [
            pltpu.VMEM((K,), jnp.int32),
            pltpu.VMEM((K,), jnp.int32),
            pltpu.VMEM((K, D), jnp.float32),
            pltpu.SemaphoreType.DMA,
            pltpu.SemaphoreType.DMA,
            pltpu.SemaphoreType.DMA,
        ]
    if with_deg:
        out_type.append(jax.ShapeDtypeStruct((NC, NUP), jnp.float32))
        scratch += [
            pltpu.VMEM_SHARED((NUP,), jnp.float32),
            pltpu.VMEM((K,), jnp.float32),
        ]

    @functools.partial(pl.kernel, mesh=_MESH, out_type=out_type,
                       scratch_types=scratch)
    def k(h_hbm, src_hbm, dst_hbm, zf_hbm, zd_hbm, ones_hbm, *rest):
        if with_deg:
            agg_out, deg_out, acc = rest[:3]
            bufs = rest[3:3 + NBUF * per_buf]
            dacc, ones_v = rest[3 + NBUF * per_buf:]
        else:
            agg_out, acc = rest[:2]
            bufs = rest[2:2 + NBUF * per_buf]
        ring = [bufs[i * per_buf:(i + 1) * per_buf] for i in range(NBUF)]
        sidx = [r[0] for r in ring]
        didx = [r[1] for r in ring]
        rows = [r[2] for r in ring]
        isem = [r[3] for r in ring]
        gs = [r[4] for r in ring]
        scs = [r[5] for r in ring]

        c = lax.axis_index("c")
        s = lax.axis_index("s")
        wid = s * NC + c
        w0 = pl.multiple_of(s * WPT, 128)

        zsem = scs[0]
        if with_deg:
            pltpu.make_async_copy(ones_hbm, ones_v, zsem).start()
            pltpu.make_async_copy(zd_hbm, dacc.at[pl.ds(w0, WPT)],
                                  zsem).start()
        for z in range(RPT // ZROWS):
            pltpu.make_async_copy(
                zf_hbm, acc.at[pl.ds(s * RPT + z * ZROWS, ZROWS)],
                zsem).start()
        def drain_zero():
            if with_deg:
                pltpu.make_async_copy(ones_hbm, ones_v, zsem).wait()
                pltpu.make_async_copy(zd_hbm, dacc.at[pl.ds(w0, WPT)],
                                      zsem).wait()
            for z in range(RPT // ZROWS):
                pltpu.make_async_copy(
                    zf_hbm, acc.at[pl.ds(s * RPT + z * ZROWS, ZROWS)],
                    zsem).wait()

        def start_idx(ci, b):
            base = pl.multiple_of(wid * EPW + ci * K, 8)
            pltpu.make_async_copy(src_hbm.at[pl.ds(base, K)], sidx[b],
                                  isem[b]).start()
            pltpu.make_async_copy(dst_hbm.at[pl.ds(base, K)], didx[b],
                                  isem[b]).start()

        def wait_idx_start_gather(b):
            pltpu.make_async_copy(src_hbm.at[pl.ds(0, K)], sidx[b],
                                  isem[b]).wait()
            pltpu.make_async_copy(dst_hbm.at[pl.ds(0, K)], didx[b],
                                  isem[b]).wait()
            pltpu.make_async_copy(h_hbm.at[sidx[b]], rows[b], gs[b]).start()

        def start_scatter(b):
            pltpu.make_async_copy(rows[b], acc.at[didx[b]],
                                  scs[b]).start(add=True)
            if with_deg:
                pltpu.make_async_copy(ones_v, dacc.at[didx[b]],
                                      scs[b]).start(add=True)

        def wait_scatter(b):
            pltpu.make_async_copy(rows[b], acc.at[didx[b]], scs[b]).wait()
            if with_deg:
                pltpu.make_async_copy(ones_v, dacc.at[didx[b]],
                                      scs[b]).wait()

        for b in range(NBUF - 1):
            start_idx(b, b)
        for b in range(GL):
            wait_idx_start_gather(b)
        drain_zero()
        plsc.subcore_barrier()

        def body(i8, carry):
            for b in range(NBUF):
                ci = NBUF * i8 + b

                @pl.when(ci < NCHUNK)
                def _():
                    @pl.when(ci + GL < NCHUNK)
                    def _():
                        wait_idx_start_gather((b + GL) % NBUF)

                    pltpu.make_async_copy(h_hbm.at[sidx[b]], rows[b],
                                          gs[b]).wait()
                    start_scatter(b)

                    @pl.when(ci >= 1)
                    def _():
                        wait_scatter((b + NBUF - 1) % NBUF)

                    @pl.when(ci + NBUF - 1 < NCHUNK)
                    def _():
                        start_idx(ci + NBUF - 1, (b + NBUF - 1) % NBUF)

            return carry

        lax.fori_loop(0, (NCHUNK + NBUF - 1) // NBUF, body, 0)
        wait_scatter((NCHUNK - 1) % NBUF)
        plsc.subcore_barrier()

        if with_deg:
            pltpu.sync_copy(dacc.at[pl.ds(w0, WPT)],
                            deg_out.at[c, pl.ds(w0, WPT)])

        @pl.when(s < NS - 1)
        def _write_full():
            pltpu.sync_copy(acc.at[pl.ds(w0, WPT)],
                            agg_out.at[c, pl.ds(w0, WPT)])

        @pl.when(s == NS - 1)
        def _write_tail():
            pltpu.sync_copy(acc.at[pl.ds(WTAIL0, N - WTAIL0)],
                            agg_out.at[c, pl.ds(WTAIL0, N - WTAIL0)])

    return k


_sc_agg_deg = _make_sc_agg(True)
_sc_agg_only = _make_sc_agg(False)


_BN = 1000


def _tc_self_body(h_ref, w_ref, b_ref, o_ref):
    o_ref[...] = (jnp.dot(h_ref[...], w_ref[...],
                          preferred_element_type=jnp.float32) + b_ref[...])


def _tc_self(h, W, b):
    return pl.pallas_call(
        _tc_self_body,
        grid=(N // _BN,),
        in_specs=[
            pl.BlockSpec((_BN, D), lambda i: (i, 0)),
            pl.BlockSpec((D, D), lambda i: (0, 0)),
            pl.BlockSpec((1, D), lambda i: (0, 0)),
        ],
        out_specs=pl.BlockSpec((_BN, D), lambda i: (i, 0)),
        out_shape=jax.ShapeDtypeStruct((N, D), jnp.float32),
    )(h, W, b.reshape(1, D))


def _tc_combine_body(relu, p_ref, d_ref, s_ref, wl_ref, o_ref):
    agg = p_ref[0] + p_ref[1]
    mean = agg / jnp.maximum(d_ref[...], 1.0)
    out = jnp.dot(mean, wl_ref[...],
                  preferred_element_type=jnp.float32) + s_ref[...]
    if relu:
        out = jnp.maximum(out, 0.0)
    o_ref[...] = out


def _tc_combine(aggp, deg_col, selfp, Wl, relu):
    return pl.pallas_call(
        functools.partial(_tc_combine_body, relu),
        grid=(N // _BN,),
        in_specs=[
            pl.BlockSpec((NC, _BN, D), lambda i: (0, i, 0)),
            pl.BlockSpec((_BN, 1), lambda i: (i, 0)),
            pl.BlockSpec((_BN, D), lambda i: (i, 0)),
            pl.BlockSpec((D, D), lambda i: (0, 0)),
        ],
        out_specs=pl.BlockSpec((_BN, D), lambda i: (i, 0)),
        out_shape=jax.ShapeDtypeStruct((N, D), jnp.float32),
    )(aggp, deg_col, selfp, Wl)


def kernel(x, edge_index, Wl0, Wr0, b0, Wl1, Wr1, b1):
    src = edge_index[0]
    dst = edge_index[1]
    zf = jnp.zeros((ZROWS, D), jnp.float32)
    zd = jnp.zeros((WPT,), jnp.float32)
    ones = jnp.ones((K,), jnp.float32)

    aggp0, degp = _sc_agg_deg(x, src, dst, zf, zd, ones)
    self0 = _tc_self(x, Wr0, b0)
    deg_col = (degp[0, :N] + degp[1, :N]).reshape(N, 1)
    h1 = _tc_combine(aggp0, deg_col, self0, Wl0, relu=True)
    (aggp1,) = _sc_agg_only(h1, src, dst, zf, zd, ones)
    self1 = _tc_self(h1, Wr1, b1)
    out = _tc_combine(aggp1, deg_col, self1, Wl1, relu=False)
    return out


# speedup vs baseline: 1.4607x; 1.0164x over previous
import functools

import jax
import jax.numpy as jnp
from jax import lax
from jax.experimental import pallas as pl
from jax.experimental.pallas import tpu as pltpu
from jax.experimental.pallas import tpu_sc as plsc

N = 10000
E = 320000
D = 128

_INFO = plsc.get_sparse_core_info()
NC = _INFO.num_cores
NS = _INFO.num_subcores
NW = NC * NS
EPW = E // NW
K = 80
NCHUNK = EPW // K
RPT = N // NS
ZROWS = 125
WPT = 640
WTAIL0 = (NS - 1) * WPT
NUP = NS * WPT

_MESH = plsc.VectorSubcoreMesh(core_axis_name="c", subcore_axis_name="s")


NBUF = 4
GL = NBUF - 2


def _make_sc_agg(with_deg):
    out_type = [jax.ShapeDtypeStruct((NC, N, D), jnp.float32)]
    scratch = [
        pltpu.VMEM_SHARED((N, D), jnp.float32),
    ]
    per_buf = 6
    for _ in range(NBUF):
        scratch += [
            pltpu.VMEM((K,), jnp.int32),
            pltpu.VMEM((K,), jnp.int32),
            pltpu.VMEM((K, D), jnp.float32),
            pltpu.SemaphoreType.DMA,
            pltpu.SemaphoreType.DMA,
            pltpu.SemaphoreType.DMA,
        ]
    if with_deg:
        out_type.append(jax.ShapeDtypeStruct((NC, NUP), jnp.float32))
        scratch += [
            pltpu.VMEM_SHARED((NUP,), jnp.float32),
            pltpu.VMEM((K,), jnp.float32),
        ]

    @functools.partial(pl.kernel, mesh=_MESH, out_type=out_type,
                       scratch_types=scratch)
    def k(h_hbm, src_hbm, dst_hbm, zf_hbm, zd_hbm, ones_hbm, *rest):
        if with_deg:
            agg_out, deg_out, acc = rest[:3]
            bufs = rest[3:3 + NBUF * per_buf]
            dacc, ones_v = rest[3 + NBUF * per_buf:]
        else:
            agg_out, acc = rest[:2]
            bufs = rest[2:2 + NBUF * per_buf]
        ring = [bufs[i * per_buf:(i + 1) * per_buf] for i in range(NBUF)]
        sidx = [r[0] for r in ring]
        didx = [r[1] for r in ring]
        rows = [r[2] for r in ring]
        isem = [r[3] for r in ring]
        gs = [r[4] for r in ring]
        scs = [r[5] for r in ring]

        c = lax.axis_index("c")
        s = lax.axis_index("s")
        wid = s * NC + c
        w0 = pl.multiple_of(s * WPT, 128)

        zsem = scs[0]
        if with_deg:
            pltpu.make_async_copy(ones_hbm, ones_v, zsem).start()
            pltpu.make_async_copy(zd_hbm, dacc.at[pl.ds(w0, WPT)],
                                  zsem).start()
        for z in range(RPT // ZROWS):
            pltpu.make_async_copy(
                zf_hbm, acc.at[pl.ds(s * RPT + z * ZROWS, ZROWS)],
                zsem).start()
        def drain_zero():
            if with_deg:
                pltpu.make_async_copy(ones_hbm, ones_v, zsem).wait()
                pltpu.make_async_copy(zd_hbm, dacc.at[pl.ds(w0, WPT)],
                                      zsem).wait()
            for z in range(RPT // ZROWS):
                pltpu.make_async_copy(
                    zf_hbm, acc.at[pl.ds(s * RPT + z * ZROWS, ZROWS)],
                    zsem).wait()

        def start_idx(ci, b):
            base = pl.multiple_of(wid * EPW + ci * K, 8)
            pltpu.make_async_copy(src_hbm.at[pl.ds(base, K)], sidx[b],
                                  isem[b]).start()
            pltpu.make_async_copy(dst_hbm.at[pl.ds(base, K)], didx[b],
                                  isem[b]).start()

        def wait_idx_start_gather(b):
            pltpu.make_async_copy(src_hbm.at[pl.ds(0, K)], sidx[b],
                                  isem[b]).wait()
            pltpu.make_async_copy(dst_hbm.at[pl.ds(0, K)], didx[b],
                                  isem[b]).wait()
            pltpu.make_async_copy(h_hbm.at[sidx[b]], rows[b], gs[b]).start()

        def start_scatter(b):
            pltpu.make_async_copy(rows[b], acc.at[didx[b]],
                                  scs[b]).start(add=True)
            if with_deg:
                pltpu.make_async_copy(ones_v, dacc.at[didx[b]],
                                      scs[b]).start(add=True)

        def wait_scatter(b):
            pltpu.make_async_copy(rows[b], acc.at[didx[b]], scs[b]).wait()
            if with_deg:
                pltpu.make_async_copy(ones_v, dacc.at[didx[b]],
                                      scs[b]).wait()

        for b in range(NBUF - 1):
            start_idx(b, b)
        for b in range(GL):
            wait_idx_start_gather(b)
        drain_zero()
        plsc.subcore_barrier()

        def body(i8, carry):
            for b in range(NBUF):
                ci = NBUF * i8 + b

                @pl.when(ci < NCHUNK)
                def _():
                    @pl.when(ci + GL < NCHUNK)
                    def _():
                        wait_idx_start_gather((b + GL) % NBUF)

                    pltpu.make_async_copy(h_hbm.at[sidx[b]], rows[b],
                                          gs[b]).wait()
                    start_scatter(b)

                    @pl.when(ci >= 1)
                    def _():
                        wait_scatter((b + NBUF - 1) % NBUF)

                    @pl.when(ci + NBUF - 1 < NCHUNK)
                    def _():
                        start_idx(ci + NBUF - 1, (b + NBUF - 1) % NBUF)

            return carry

        lax.fori_loop(0, (NCHUNK + NBUF - 1) // NBUF, body, 0)
        wait_scatter((NCHUNK - 1) % NBUF)
        plsc.subcore_barrier()

        if with_deg:
            pltpu.sync_copy(dacc.at[pl.ds(w0, WPT)],
                            deg_out.at[c, pl.ds(w0, WPT)])

        @pl.when(s < NS - 1)
        def _write_full():
            pltpu.sync_copy(acc.at[pl.ds(w0, WPT)],
                            agg_out.at[c, pl.ds(w0, WPT)])

        @pl.when(s == NS - 1)
        def _write_tail():
            pltpu.sync_copy(acc.at[pl.ds(WTAIL0, N - WTAIL0)],
                            agg_out.at[c, pl.ds(WTAIL0, N - WTAIL0)])

    return k


_sc_agg_deg = _make_sc_agg(True)
_sc_agg_only = _make_sc_agg(False)


_BN = 2000


def _tc_self_body(h_ref, w_ref, b_ref, o_ref):
    o_ref[...] = (jnp.dot(h_ref[...], w_ref[...],
                          preferred_element_type=jnp.float32) + b_ref[...])


def _tc_self(h, W, b):
    return pl.pallas_call(
        _tc_self_body,
        grid=(N // _BN,),
        in_specs=[
            pl.BlockSpec((_BN, D), lambda i: (i, 0)),
            pl.BlockSpec((D, D), lambda i: (0, 0)),
            pl.BlockSpec((1, D), lambda i: (0, 0)),
        ],
        out_specs=pl.BlockSpec((_BN, D), lambda i: (i, 0)),
        out_shape=jax.ShapeDtypeStruct((N, D), jnp.float32),
    )(h, W, b.reshape(1, D))


def _tc_combine_body(relu, p_ref, d_ref, s_ref, wl_ref, o_ref):
    agg = p_ref[0] + p_ref[1]
    mean = agg / jnp.maximum(d_ref[...], 1.0)
    out = jnp.dot(mean, wl_ref[...],
                  preferred_element_type=jnp.float32) + s_ref[...]
    if relu:
        out = jnp.maximum(out, 0.0)
    o_ref[...] = out


def _tc_combine(aggp, deg_col, selfp, Wl, relu):
    return pl.pallas_call(
        functools.partial(_tc_combine_body, relu),
        grid=(N // _BN,),
        in_specs=[
            pl.BlockSpec((NC, _BN, D), lambda i: (0, i, 0)),
            pl.BlockSpec((_BN, 1), lambda i: (i, 0)),
            pl.BlockSpec((_BN, D), lambda i: (i, 0)),
            pl.BlockSpec((D, D), lambda i: (0, 0)),
        ],
        out_specs=pl.BlockSpec((_BN, D), lambda i: (i, 0)),
        out_shape=jax.ShapeDtypeStruct((N, D), jnp.float32),
    )(aggp, deg_col, selfp, Wl)


def kernel(x, edge_index, Wl0, Wr0, b0, Wl1, Wr1, b1):
    src = edge_index[0]
    dst = edge_index[1]
    zf = jnp.zeros((ZROWS, D), jnp.float32)
    zd = jnp.zeros((WPT,), jnp.float32)
    ones = jnp.ones((K,), jnp.float32)

    aggp0, degp = _sc_agg_deg(x, src, dst, zf, zd, ones)
    self0 = _tc_self(x, Wr0, b0)
    deg_col = (degp[0, :N] + degp[1, :N]).reshape(N, 1)
    h1 = _tc_combine(aggp0, deg_col, self0, Wl0, relu=True)
    (aggp1,) = _sc_agg_only(h1, src, dst, zf, zd, ones)
    self1 = _tc_self(h1, Wr1, b1)
    out = _tc_combine(aggp1, deg_col, self1, Wl1, relu=False)
    return out


# speedup vs baseline: 1.4632x; 1.0017x over previous
import functools

import jax
import jax.numpy as jnp
from jax import lax
from jax.experimental import pallas as pl
from jax.experimental.pallas import tpu as pltpu
from jax.experimental.pallas import tpu_sc as plsc

N = 10000
E = 320000
D = 128

_INFO = plsc.get_sparse_core_info()
NC = _INFO.num_cores
NS = _INFO.num_subcores
NW = NC * NS
EPW = E // NW
K = 80
NCHUNK = EPW // K
RPT = N // NS
ZROWS = 125
WPT = 640
WTAIL0 = (NS - 1) * WPT
NUP = NS * WPT

_MESH = plsc.VectorSubcoreMesh(core_axis_name="c", subcore_axis_name="s")


NBUF = 4
GL = NBUF - 2


def _make_sc_agg(with_deg):
    out_type = [jax.ShapeDtypeStruct((NC, N, D), jnp.float32)]
    scratch = [
        pltpu.VMEM_SHARED((N, D), jnp.float32),
    ]
    per_buf = 6
    for _ in range(NBUF):
        scratch += [
            pltpu.VMEM((K,), jnp.int32),
            pltpu.VMEM((K,), jnp.int32),
            pltpu.VMEM((K, D), jnp.float32),
            pltpu.SemaphoreType.DMA,
            pltpu.SemaphoreType.DMA,
            pltpu.SemaphoreType.DMA,
        ]
    if with_deg:
        out_type.append(jax.ShapeDtypeStruct((NC, NUP), jnp.float32))
        scratch += [
            pltpu.VMEM_SHARED((NUP,), jnp.float32),
            pltpu.VMEM((K,), jnp.float32),
        ]

    @functools.partial(pl.kernel, mesh=_MESH, out_type=out_type,
                       scratch_types=scratch)
    def k(h_hbm, src_hbm, dst_hbm, zf_hbm, zd_hbm, ones_hbm, *rest):
        if with_deg:
            agg_out, deg_out, acc = rest[:3]
            bufs = rest[3:3 + NBUF * per_buf]
            dacc, ones_v = rest[3 + NBUF * per_buf:]
        else:
            agg_out, acc = rest[:2]
            bufs = rest[2:2 + NBUF * per_buf]
        ring = [bufs[i * per_buf:(i + 1) * per_buf] for i in range(NBUF)]
        sidx = [r[0] for r in ring]
        didx = [r[1] for r in ring]
        rows = [r[2] for r in ring]
        isem = [r[3] for r in ring]
        gs = [r[4] for r in ring]
        scs = [r[5] for r in ring]

        c = lax.axis_index("c")
        s = lax.axis_index("s")
        wid = s * NC + c
        w0 = pl.multiple_of(s * WPT, 128)

        zsem = scs[0]
        if with_deg:
            pltpu.make_async_copy(ones_hbm, ones_v, zsem).start()
            pltpu.make_async_copy(zd_hbm, dacc.at[pl.ds(w0, WPT)],
                                  zsem).start()
        for z in range(RPT // ZROWS):
            pltpu.make_async_copy(
                zf_hbm, acc.at[pl.ds(s * RPT + z * ZROWS, ZROWS)],
                zsem).start()
        def drain_zero():
            if with_deg:
                pltpu.make_async_copy(ones_hbm, ones_v, zsem).wait()
                pltpu.make_async_copy(zd_hbm, dacc.at[pl.ds(w0, WPT)],
                                      zsem).wait()
            for z in range(RPT // ZROWS):
                pltpu.make_async_copy(
                    zf_hbm, acc.at[pl.ds(s * RPT + z * ZROWS, ZROWS)],
                    zsem).wait()

        def start_idx(ci, b):
            base = pl.multiple_of(wid * EPW + ci * K, 8)
            pltpu.make_async_copy(src_hbm.at[pl.ds(base, K)], sidx[b],
                                  isem[b]).start()
            pltpu.make_async_copy(dst_hbm.at[pl.ds(base, K)], didx[b],
                                  isem[b]).start()

        def wait_idx_start_gather(b):
            pltpu.make_async_copy(src_hbm.at[pl.ds(0, K)], sidx[b],
                                  isem[b]).wait()
            pltpu.make_async_copy(dst_hbm.at[pl.ds(0, K)], didx[b],
                                  isem[b]).wait()
            pltpu.make_async_copy(h_hbm.at[sidx[b]], rows[b], gs[b]).start()

        def start_scatter(b):
            pltpu.make_async_copy(rows[b], acc.at[didx[b]],
                                  scs[b]).start(add=True)
            if with_deg:
                pltpu.make_async_copy(ones_v, dacc.at[didx[b]],
                                      scs[b]).start(add=True)

        def wait_scatter(b):
            pltpu.make_async_copy(rows[b], acc.at[didx[b]], scs[b]).wait()
            if with_deg:
                pltpu.make_async_copy(ones_v, dacc.at[didx[b]],
                                      scs[b]).wait()

        for b in range(NBUF - 1):
            start_idx(b, b)
        for b in range(GL):
            wait_idx_start_gather(b)
        drain_zero()
        plsc.subcore_barrier()

        def body(i8, carry):
            for b in range(NBUF):
                ci = NBUF * i8 + b

                @pl.when(ci < NCHUNK)
                def _():
                    @pl.when(ci + GL < NCHUNK)
                    def _():
                        wait_idx_start_gather((b + GL) % NBUF)

                    pltpu.make_async_copy(h_hbm.at[sidx[b]], rows[b],
                                          gs[b]).wait()
                    start_scatter(b)

                    @pl.when(ci >= 1)
                    def _():
                        wait_scatter((b + NBUF - 1) % NBUF)

                    @pl.when(ci + NBUF - 1 < NCHUNK)
                    def _():
                        start_idx(ci + NBUF - 1, (b + NBUF - 1) % NBUF)

            return carry

        lax.fori_loop(0, (NCHUNK + NBUF - 1) // NBUF, body, 0)
        wait_scatter((NCHUNK - 1) % NBUF)
        plsc.subcore_barrier()

        if with_deg:
            pltpu.sync_copy(dacc.at[pl.ds(w0, WPT)],
                            deg_out.at[c, pl.ds(w0, WPT)])

        @pl.when(s < NS - 1)
        def _write_full():
            pltpu.sync_copy(acc.at[pl.ds(w0, WPT)],
                            agg_out.at[c, pl.ds(w0, WPT)])

        @pl.when(s == NS - 1)
        def _write_tail():
            pltpu.sync_copy(acc.at[pl.ds(WTAIL0, N - WTAIL0)],
                            agg_out.at[c, pl.ds(WTAIL0, N - WTAIL0)])

    return k


_sc_agg_deg = _make_sc_agg(True)
_sc_agg_only = _make_sc_agg(False)


_BN = 10000


def _tc_self_body(h_ref, w_ref, b_ref, o_ref):
    o_ref[...] = (jnp.dot(h_ref[...], w_ref[...],
                          preferred_element_type=jnp.float32) + b_ref[...])


def _tc_self(h, W, b):
    return pl.pallas_call(
        _tc_self_body,
        grid=(N // _BN,),
        in_specs=[
            pl.BlockSpec((_BN, D), lambda i: (i, 0)),
            pl.BlockSpec((D, D), lambda i: (0, 0)),
            pl.BlockSpec((1, D), lambda i: (0, 0)),
        ],
        out_specs=pl.BlockSpec((_BN, D), lambda i: (i, 0)),
        out_shape=jax.ShapeDtypeStruct((N, D), jnp.float32),
    )(h, W, b.reshape(1, D))


def _tc_combine_body(relu, p_ref, d_ref, s_ref, wl_ref, o_ref):
    agg = p_ref[0] + p_ref[1]
    mean = agg / jnp.maximum(d_ref[...], 1.0)
    out = jnp.dot(mean, wl_ref[...],
                  preferred_element_type=jnp.float32) + s_ref[...]
    if relu:
        out = jnp.maximum(out, 0.0)
    o_ref[...] = out


def _tc_combine(aggp, deg_col, selfp, Wl, relu):
    return pl.pallas_call(
        functools.partial(_tc_combine_body, relu),
        grid=(N // _BN,),
        in_specs=[
            pl.BlockSpec((NC, _BN, D), lambda i: (0, i, 0)),
            pl.BlockSpec((_BN, 1), lambda i: (i, 0)),
            pl.BlockSpec((_BN, D), lambda i: (i, 0)),
            pl.BlockSpec((D, D), lambda i: (0, 0)),
        ],
        out_specs=pl.BlockSpec((_BN, D), lambda i: (i, 0)),
        out_shape=jax.ShapeDtypeStruct((N, D), jnp.float32),
    )(aggp, deg_col, selfp, Wl)


def kernel(x, edge_index, Wl0, Wr0, b0, Wl1, Wr1, b1):
    src = edge_index[0]
    dst = edge_index[1]
    zf = jnp.zeros((ZROWS, D), jnp.float32)
    zd = jnp.zeros((WPT,), jnp.float32)
    ones = jnp.ones((K,), jnp.float32)

    aggp0, degp = _sc_agg_deg(x, src, dst, zf, zd, ones)
    self0 = _tc_self(x, Wr0, b0)
    deg_col = (degp[0, :N] + degp[1, :N]).reshape(N, 1)
    h1 = _tc_combine(aggp0, deg_col, self0, Wl0, relu=True)
    (aggp1,) = _sc_agg_only(h1, src, dst, zf, zd, ones)
    self1 = _tc_self(h1, Wr1, b1)
    out = _tc_combine(aggp1, deg_col, self1, Wl1, relu=False)
    return out
